# baseline (speedup 1.0000x reference)
import jax
import jax.numpy as jnp
from jax import lax
from jax.experimental import pallas as pl
from jax.experimental.pallas import tpu as pltpu
from jax.experimental.pallas import tpu_sc as plsc

BATCH = 4096
HIST = 50
DIM = 128

NC = 2
NS = 16
NW = NC * NS

K = 4
EL_PER_PART = BATCH // K
EL_PER_W = EL_PER_PART // NW
EL_PER_CHUNK = 2
CHUNK = EL_PER_CHUNK * HIST
N_CHUNKS = EL_PER_W // EL_PER_CHUNK
NBUF = 4


def _emb_kernel(idx_hbm, table_hbm, out_hbm, idx_v, rows_v, gsems, osems):
    wid = lax.axis_index("s") * NC + lax.axis_index("c")
    ebase = wid * EL_PER_W

    pltpu.sync_copy(idx_hbm.at[wid], idx_v)

    for b in range(NBUF):
        pltpu.async_copy(table_hbm.at[idx_v.at[b]], rows_v.at[b], gsems.at[b])

    @pl.loop(0, N_CHUNKS, step=NBUF)
    def _group(g):
        for b in range(NBUF):
            j = g + b
            pltpu.make_async_copy(table_hbm.at[idx_v.at[0]], rows_v.at[b],
                                  gsems.at[b]).wait()
            e = ebase + j * EL_PER_CHUNK
            pltpu.async_copy(rows_v.at[b, pl.ds(0, HIST)], out_hbm.at[e],
                             osems.at[b])
            pltpu.async_copy(rows_v.at[b, pl.ds(HIST, HIST)],
                             out_hbm.at[e + 1], osems.at[b])

            @pl.when(j + NBUF < N_CHUNKS)
            def _():
                pltpu.make_async_copy(rows_v.at[b, pl.ds(0, HIST)],
                                      out_hbm.at[0], osems.at[b]).wait()
                pltpu.make_async_copy(rows_v.at[b, pl.ds(0, HIST)],
                                      out_hbm.at[0], osems.at[b]).wait()
                pltpu.async_copy(table_hbm.at[idx_v.at[j + NBUF]],
                                 rows_v.at[b], gsems.at[b])

    for b in range(NBUF):
        pltpu.make_async_copy(rows_v.at[b, pl.ds(0, HIST)], out_hbm.at[0],
                              osems.at[b]).wait()
        pltpu.make_async_copy(rows_v.at[b, pl.ds(0, HIST)], out_hbm.at[0],
                              osems.at[b]).wait()


@jax.jit
def kernel(token_ids, weight):
    idx = token_ids.astype(jnp.int32).reshape(K, NW, N_CHUNKS, CHUNK)
    mesh = plsc.VectorSubcoreMesh(core_axis_name="c", subcore_axis_name="s",
                                  num_cores=NC, num_subcores=NS)
    part = pl.kernel(
        _emb_kernel,
        out_type=jax.ShapeDtypeStruct((EL_PER_PART, HIST, DIM), jnp.float32),
        mesh=mesh,
        scratch_types=[
            pltpu.VMEM((N_CHUNKS, CHUNK), jnp.int32),
            pltpu.VMEM((NBUF, CHUNK, DIM), jnp.float32),
            pltpu.SemaphoreType.DMA((NBUF,)),
            pltpu.SemaphoreType.DMA((NBUF,)),
        ],
    )
    return jnp.concatenate([part(idx[p], weight) for p in range(K)], axis=0)


# speedup vs baseline: 3.1789x; 3.1789x over previous
import jax
import jax.numpy as jnp
from jax import lax
from jax.experimental import pallas as pl
from jax.experimental.pallas import tpu as pltpu
from jax.experimental.pallas import tpu_sc as plsc

BATCH = 4096
HIST = 50
DIM = 128

NC = 2
NS = 16
NW = NC * NS

TOTAL = BATCH * HIST
CHUNK = 128
PER_W = TOTAL // NW
N_CHUNKS = PER_W // CHUNK
NBUF = 5


def _emb_kernel(idx_hbm, table_hbm, out_hbm, idx_v, rows_v, gsems, osems):
    wid = lax.axis_index("s") * NC + lax.axis_index("c")
    base = wid * PER_W

    pltpu.sync_copy(idx_hbm.at[wid], idx_v)

    for b in range(NBUF):
        pltpu.async_copy(table_hbm.at[idx_v.at[b]], rows_v.at[b], gsems.at[b])

    @pl.loop(0, N_CHUNKS, step=NBUF)
    def _group(g):
        for b in range(NBUF):
            j = g + b
            pltpu.make_async_copy(table_hbm.at[idx_v.at[0]], rows_v.at[b],
                                  gsems.at[b]).wait()
            pltpu.async_copy(rows_v.at[b],
                             out_hbm.at[pl.ds(base + j * CHUNK, CHUNK)],
                             osems.at[b])

            @pl.when(j + NBUF < N_CHUNKS)
            def _():
                pltpu.make_async_copy(rows_v.at[b],
                                      out_hbm.at[pl.ds(0, CHUNK)],
                                      osems.at[b]).wait()
                pltpu.async_copy(table_hbm.at[idx_v.at[j + NBUF]],
                                 rows_v.at[b], gsems.at[b])

    for b in range(NBUF):
        pltpu.make_async_copy(rows_v.at[b], out_hbm.at[pl.ds(0, CHUNK)],
                              osems.at[b]).wait()


@jax.jit
def kernel(token_ids, weight):
    idx = token_ids.astype(jnp.int32).T.reshape(NW, N_CHUNKS, CHUNK)
    mesh = plsc.VectorSubcoreMesh(core_axis_name="c", subcore_axis_name="s",
                                  num_cores=NC, num_subcores=NS)
    out = pl.kernel(
        _emb_kernel,
        out_type=jax.ShapeDtypeStruct((TOTAL, DIM), jnp.float32),
        mesh=mesh,
        scratch_types=[
            pltpu.VMEM((N_CHUNKS, CHUNK), jnp.int32),
            pltpu.VMEM((NBUF, CHUNK, DIM), jnp.float32),
            pltpu.SemaphoreType.DMA((NBUF,)),
            pltpu.SemaphoreType.DMA((NBUF,)),
        ],
    )(idx, weight)
    return out.reshape(HIST, BATCH, DIM).transpose(1, 0, 2)


# speedup vs baseline: 3.2685x; 1.0282x over previous
import jax
import jax.numpy as jnp
from jax import lax
from jax.experimental import pallas as pl
from jax.experimental.pallas import tpu as pltpu
from jax.experimental.pallas import tpu_sc as plsc

BATCH = 4096
HIST = 50
DIM = 128

NC = 2
NS = 16
NW = NC * NS

TOTAL = BATCH * HIST
CHUNK = 128
N_CHUNKS = HIST
NBUF = 5


def _emb_kernel(idx_hbm, table_hbm, out_hbm, idx_v, rows_v, gsems, osems):
    wid = lax.axis_index("s") * NC + lax.axis_index("c")
    base = wid * CHUNK

    pltpu.sync_copy(idx_hbm.at[pl.ds(0, HIST), pl.ds(wid * CHUNK, CHUNK)],
                    idx_v)

    for b in range(NBUF):
        pltpu.async_copy(table_hbm.at[idx_v.at[b]], rows_v.at[b], gsems.at[b])

    @pl.loop(0, N_CHUNKS, step=NBUF)
    def _group(g):
        for b in range(NBUF):
            j = g + b
            pltpu.make_async_copy(table_hbm.at[idx_v.at[0]], rows_v.at[b],
                                  gsems.at[b]).wait()
            pltpu.async_copy(rows_v.at[b],
                             out_hbm.at[pl.ds(j * BATCH + base, CHUNK)],
                             osems.at[b])

            @pl.when(j + NBUF < N_CHUNKS)
            def _():
                pltpu.make_async_copy(rows_v.at[b],
                                      out_hbm.at[pl.ds(0, CHUNK)],
                                      osems.at[b]).wait()
                pltpu.async_copy(table_hbm.at[idx_v.at[j + NBUF]],
                                 rows_v.at[b], gsems.at[b])

    for b in range(NBUF):
        pltpu.make_async_copy(rows_v.at[b], out_hbm.at[pl.ds(0, CHUNK)],
                              osems.at[b]).wait()


@jax.jit
def kernel(token_ids, weight):
    idx = token_ids.astype(jnp.int32).T
    mesh = plsc.VectorSubcoreMesh(core_axis_name="c", subcore_axis_name="s",
                                  num_cores=NC, num_subcores=NS)
    out = pl.kernel(
        _emb_kernel,
        out_type=jax.ShapeDtypeStruct((TOTAL, DIM), jnp.float32),
        mesh=mesh,
        scratch_types=[
            pltpu.VMEM((N_CHUNKS, CHUNK), jnp.int32),
            pltpu.VMEM((NBUF, CHUNK, DIM), jnp.float32),
            pltpu.SemaphoreType.DMA((NBUF,)),
            pltpu.SemaphoreType.DMA((NBUF,)),
        ],
    )(idx, weight)
    return out.reshape(HIST, BATCH, DIM).transpose(1, 0, 2)
